# baseline (speedup 1.0000x reference)
import jax
import jax.numpy as jnp
from jax import lax
from jax.experimental import pallas as pl
from jax.experimental.pallas import tpu as pltpu
from jax.experimental.pallas import tpu_sc as plsc

_N_COMP, _N_NODES, _N_T = 16, 325, 12
_W = _N_COMP * _N_NODES * _N_T
_B = 64
_NC, _NS, _L = 2, 16, 16
_NW = _NC * _NS
_BPW = _B // _NW
_NCH = 6
_CW = _W // _NCH
_NT = _BPW * _NCH
_NB = 3


def _body(warr_hbm, h_hbm, d_hbm, out_hbm,
          ibuf, hbuf, dbuf, obuf, hsem, dsem, osem):
    wid = lax.axis_index("s") * _NC + lax.axis_index("c")
    pltpu.sync_copy(warr_hbm.at[wid], ibuf)

    def start_gathers(t):
        s = t % _NB
        ch = pltpu.async_copy(
            h_hbm.at[ibuf.at[pl.ds(t * 8, 1)]],
            hbuf.at[pl.ds(s, 1)], hsem.at[s])
        cd = pltpu.async_copy(
            d_hbm.at[ibuf.at[pl.ds((_NT + t) * 8, 1)]],
            dbuf.at[pl.ds(s, 1)], dsem.at[s])
        return ch, cd

    copies = {}
    ocopies = {}
    for t in range(_NB):
        copies[t] = start_gathers(t)

    for t in range(_NT):
        s = t % _NB
        ch, cd = copies.pop(t)
        ch.wait()
        cd.wait()
        if t >= _NB:
            ocopies.pop(t - _NB).wait()

        def _add(i, carry):
            sl = pl.ds(i * _L, _L)
            obuf[s, sl] = hbuf[s, sl] + dbuf[s, sl]
            return carry

        lax.fori_loop(0, _CW // _L, _add, 0)
        if t + _NB < _NT:
            copies[t + _NB] = start_gathers(t + _NB)
        ocopies[t] = pltpu.async_copy(
            obuf.at[pl.ds(s, 1)],
            out_hbm.at[pl.ds(wid * _NT + t, 1)], osem.at[s])

    for t in sorted(ocopies):
        ocopies.pop(t).wait()


@jax.jit
def _run(H, D, h_ebd, d_ebd):
    w = jnp.arange(_NW)
    t = jnp.arange(_NT)
    r, c = t // _NCH, t % _NCH
    bidx = 2 * w[:, None] + r[None, :]
    hslots = H[bidx].astype(jnp.int32) * _NCH + c[None, :]
    dslots = D[bidx].astype(jnp.int32) * _NCH + c[None, :]
    slots = jnp.concatenate([hslots, dslots], axis=1)
    warr = jnp.broadcast_to(slots[:, :, None], (_NW, 2 * _NT, 8))
    warr = warr.reshape(_NW, 2 * _NT * 8)

    h2 = h_ebd.reshape(24 * _NCH, _CW)
    d2 = d_ebd.reshape(7 * _NCH, _CW)
    mesh = plsc.VectorSubcoreMesh(core_axis_name="c", subcore_axis_name="s")
    out = pl.kernel(
        _body,
        out_type=jax.ShapeDtypeStruct((_B * _NCH, _CW), jnp.float32),
        mesh=mesh,
        compiler_params=pltpu.CompilerParams(use_tc_tiling_on_sc=False),
        scratch_types=[
            pltpu.VMEM((2 * _NT * 8,), jnp.int32),
            pltpu.VMEM((_NB, _CW), jnp.float32),
            pltpu.VMEM((_NB, _CW), jnp.float32),
            pltpu.VMEM((_NB, _CW), jnp.float32),
            pltpu.SemaphoreType.DMA((_NB,)),
            pltpu.SemaphoreType.DMA((_NB,)),
            pltpu.SemaphoreType.DMA((_NB,)),
        ],
    )(warr, h2, d2)
    return out.reshape(_B, _N_COMP, _N_NODES, _N_T)


def kernel(H, D, h_ebd, d_ebd):
    return _run(H, D, h_ebd, d_ebd)


# speedup vs baseline: 1.6174x; 1.6174x over previous
import jax
import jax.numpy as jnp
from jax import lax
from jax.experimental import pallas as pl
from jax.experimental.pallas import tpu as pltpu
from jax.experimental.pallas import tpu_sc as plsc

_N_COMP, _N_NODES, _N_T = 16, 325, 12
_W = _N_COMP * _N_NODES * _N_T
_B = 64
_NC, _NS, _L = 2, 16, 16
_NW = _NC * _NS
_BPW = _B // _NW


def _body(hidx_hbm, didx_hbm, h_hbm, d_hbm, out_hbm,
          hidx_v, didx_v, hrow, drow, sem_h, sem_d):
    wid = lax.axis_index("s") * _NC + lax.axis_index("c")
    for r in range(_BPW):
        b = wid * _BPW + r
        pltpu.sync_copy(hidx_hbm.at[b], hidx_v)
        pltpu.sync_copy(didx_hbm.at[b], didx_v)
        cp_h = pltpu.async_copy(
            h_hbm.at[hidx_v.at[pl.ds(0, 1)]], hrow, sem_h)
        cp_d = pltpu.async_copy(
            d_hbm.at[didx_v.at[pl.ds(0, 1)]], drow, sem_d)
        cp_h.wait()
        cp_d.wait()

        def _add(i, carry):
            sl = pl.ds(i * _L, _L)
            hrow[0, sl] = hrow[0, sl] + drow[0, sl]
            return carry

        lax.fori_loop(0, _W // _L, _add, 0)
        pltpu.sync_copy(hrow, out_hbm.at[pl.ds(b, 1)])


@jax.jit
def _run(H, D, h_ebd, d_ebd):
    hidx = jnp.broadcast_to(H[:, None], (_B, _L)).astype(jnp.int32)
    didx = jnp.broadcast_to(D[:, None], (_B, _L)).astype(jnp.int32)
    mesh = plsc.VectorSubcoreMesh(core_axis_name="c", subcore_axis_name="s")
    out = pl.kernel(
        _body,
        out_type=jax.ShapeDtypeStruct((_B, _W), jnp.float32),
        mesh=mesh,
        compiler_params=pltpu.CompilerParams(use_tc_tiling_on_sc=False),
        scratch_types=[
            pltpu.VMEM((_L,), jnp.int32),
            pltpu.VMEM((_L,), jnp.int32),
            pltpu.VMEM((1, _W), jnp.float32),
            pltpu.VMEM((1, _W), jnp.float32),
            pltpu.SemaphoreType.DMA,
            pltpu.SemaphoreType.DMA,
        ],
    )(hidx, didx, h_ebd, d_ebd)
    return out.reshape(_B, _N_COMP, _N_NODES, _N_T)


def kernel(H, D, h_ebd, d_ebd):
    return _run(H, D, h_ebd, d_ebd)


# speedup vs baseline: 1.7570x; 1.0863x over previous
import jax
import jax.numpy as jnp
from jax import lax
from jax.experimental import pallas as pl
from jax.experimental.pallas import tpu as pltpu
from jax.experimental.pallas import tpu_sc as plsc

_N_COMP, _N_NODES, _N_T = 16, 325, 12
_W = _N_COMP * _N_NODES * _N_T
_B = 64
_NC, _NS, _L = 2, 16, 16
_NW = _NC * _NS
_BPW = _B // _NW


def _body(hidx_hbm, didx_hbm, h_hbm, d_hbm, out_hbm,
          hidx_v, didx_v, hrow, drow, sem_h, sem_d):
    wid = lax.axis_index("s") * _NC + lax.axis_index("c")
    for r in range(_BPW):
        b = wid * _BPW + r
        pltpu.sync_copy(hidx_hbm.at[b], hidx_v)
        pltpu.sync_copy(didx_hbm.at[b], didx_v)
        hi = jnp.max(hidx_v[...])
        di = jnp.max(didx_v[...])
        cp_h = pltpu.async_copy(h_hbm.at[pl.ds(hi, 1)], hrow, sem_h)
        cp_d = pltpu.async_copy(d_hbm.at[pl.ds(di, 1)], drow, sem_d)
        cp_h.wait()
        cp_d.wait()

        def _add(i, carry):
            sl = pl.ds(i * _L, _L)
            hrow[0, sl] = hrow[0, sl] + drow[0, sl]
            return carry

        lax.fori_loop(0, _W // _L, _add, 0)
        pltpu.sync_copy(hrow, out_hbm.at[pl.ds(b, 1)])


@jax.jit
def _run(H, D, h_ebd, d_ebd):
    hidx = jnp.broadcast_to(H[:, None], (_B, _L)).astype(jnp.int32)
    didx = jnp.broadcast_to(D[:, None], (_B, _L)).astype(jnp.int32)
    mesh = plsc.VectorSubcoreMesh(core_axis_name="c", subcore_axis_name="s")
    out = pl.kernel(
        _body,
        out_type=jax.ShapeDtypeStruct((_B, _W), jnp.float32),
        mesh=mesh,
        compiler_params=pltpu.CompilerParams(needs_layout_passes=False),
        scratch_types=[
            pltpu.VMEM((_L,), jnp.int32),
            pltpu.VMEM((_L,), jnp.int32),
            pltpu.VMEM((1, _W), jnp.float32),
            pltpu.VMEM((1, _W), jnp.float32),
            pltpu.SemaphoreType.DMA,
            pltpu.SemaphoreType.DMA,
        ],
    )(hidx, didx, h_ebd, d_ebd)
    return out.reshape(_B, _N_COMP, _N_NODES, _N_T)


def kernel(H, D, h_ebd, d_ebd):
    return _run(H, D, h_ebd, d_ebd)
